# baseline (speedup 1.0000x reference)
import functools

import jax
import jax.numpy as jnp
from jax.experimental import pallas as pl
from jax.experimental.pallas import tpu as pltpu

TAU = 0.2
TOPK = 8
ALPHA = 0.3
B = 4096
D = 128
BM = 512

_NEG = -3.0e38


def _norm_rows(x):
    ss = jnp.sum(x * x, axis=1, keepdims=True)
    return x * jax.lax.rsqrt(jnp.maximum(ss, 1e-24))


def _layer_norm(y, gamma, beta):
    mu = jnp.mean(y, axis=1, keepdims=True)
    var = jnp.mean((y - mu) * (y - mu), axis=1, keepdims=True)
    return (y - mu) * jax.lax.rsqrt(var + 1e-5) * gamma + beta


_SORT8_STAGES = (
    ((0, 1), (2, 3), (4, 5), (6, 7)),
    ((0, 2), (1, 3), (4, 6), (5, 7)),
    ((1, 2), (5, 6)),
    ((0, 4), (1, 5), (2, 6), (3, 7)),
    ((2, 4), (3, 5)),
    ((1, 2), (3, 4), (5, 6)),
)
_BITONIC8_STAGES = (
    ((0, 4), (1, 5), (2, 6), (3, 7)),
    ((0, 2), (1, 3), (4, 6), (5, 7)),
    ((0, 1), (2, 3), (4, 5), (6, 7)),
)


def _apply_net(v, stages):
    v = list(v)
    for stage in stages:
        for i, j in stage:
            hi = jnp.maximum(v[i], v[j])
            lo = jnp.minimum(v[i], v[j])
            v[i], v[j] = hi, lo
    return v


def _merge_top8(a, b):
    d = [jnp.maximum(a[i], b[7 - i]) for i in range(8)]
    return _apply_net(d, _BITONIC8_STAGES)


def _topk_softmax_msg(s, P, g_raw, gamma, beta):
    ncol = s.shape[1] // 128
    sl = [s[:, k * 128:(k + 1) * 128] for k in range(ncol)]
    groups = [_apply_net(sl[8 * k:8 * k + 8], _SORT8_STAGES)
              for k in range(ncol // 8)]
    while len(groups) > 1:
        groups = [_merge_top8(groups[2 * k], groups[2 * k + 1])
                  for k in range(len(groups) // 2)]
    d = groups[0]
    tops = []
    for _ in range(TOPK - 1):
        m = jnp.max(d[0], axis=1, keepdims=True)
        tops.append(m)
        eq = d[0] == m
        for i in range(TOPK - 1):
            d[i] = jnp.where(eq, d[i + 1], d[i])
        d[TOPK - 1] = jnp.where(eq, _NEG, d[TOPK - 1])
    thr = jnp.max(d[0], axis=1, keepdims=True)
    tops.append(thr)
    e = jnp.where(s >= thr, jnp.exp(s.astype(jnp.bfloat16)),
                  jnp.bfloat16(0.0))
    z = sum(jnp.exp(m.astype(jnp.bfloat16).astype(jnp.float32))
            for m in tops)
    msg = jax.lax.dot_general(
        e, P,
        (((1,), (0,)), ((), ())),
        preferred_element_type=jnp.float32,
    ) / z
    return _layer_norm(g_raw + ALPHA * msg, gamma, beta)


def _prologue(gI_ref, gT_ref, W_i_ref, W_t_ref,
              gin_ref, gtn_ref, Pi_ref, Pt_ref):
    gI = gI_ref[...]
    gT = gT_ref[...]
    gin_ref[...] = _norm_rows(gI).astype(jnp.bfloat16)
    gtn_ref[...] = _norm_rows(gT).astype(jnp.bfloat16)
    Pi_ref[...] = jax.lax.dot_general(
        gI.astype(jnp.bfloat16), W_i_ref[...].astype(jnp.bfloat16),
        (((1,), (1,)), ((), ())),
        preferred_element_type=jnp.float32).astype(jnp.bfloat16)
    Pt_ref[...] = jax.lax.dot_general(
        gT.astype(jnp.bfloat16), W_t_ref[...].astype(jnp.bfloat16),
        (((1,), (1,)), ((), ())),
        preferred_element_type=jnp.float32).astype(jnp.bfloat16)


def _body(gI_blk, gT_blk, gin_ref, gtn_ref, Pi_ref, Pt_ref,
          ln_i_g, ln_i_b, ln_t_g, ln_t_b,
          S_ref, gI2_ref, gT2_ref):
    gin = gin_ref
    gtn = gtn_ref
    Pi = Pi_ref
    Pt = Pt_ref
    inv_tau = jnp.float32(1.0 / TAU)

    gib = _norm_rows(gI_blk[...]).astype(jnp.bfloat16)
    s = jax.lax.dot_general(
        gib, gtn[...],
        (((1,), (1,)), ((), ())),
        preferred_element_type=jnp.float32) * inv_tau
    S_ref[...] = s
    gI2_ref[...] = _topk_softmax_msg(
        s, Pt[...], gI_blk[...], ln_i_g[...], ln_i_b[...])

    gtb = _norm_rows(gT_blk[...]).astype(jnp.bfloat16)
    st = jax.lax.dot_general(
        gtb, gin[...],
        (((1,), (1,)), ((), ())),
        preferred_element_type=jnp.float32) * inv_tau
    gT2_ref[...] = _topk_softmax_msg(
        st, Pi[...], gT_blk[...], ln_t_g[...], ln_t_b[...])


@jax.jit
def kernel(gI, gT, W_i, W_t, ln_i_g, ln_i_b, ln_t_g, ln_t_b):
    full = lambda *_: (0, 0)
    gin, gtn, Pi, Pt = pl.pallas_call(
        _prologue,
        in_specs=[
            pl.BlockSpec((B, D), full),
            pl.BlockSpec((B, D), full),
            pl.BlockSpec((D, D), full),
            pl.BlockSpec((D, D), full),
        ],
        out_specs=(
            pl.BlockSpec((B, D), full),
            pl.BlockSpec((B, D), full),
            pl.BlockSpec((B, D), full),
            pl.BlockSpec((B, D), full),
        ),
        out_shape=(
            jax.ShapeDtypeStruct((B, D), jnp.bfloat16),
            jax.ShapeDtypeStruct((B, D), jnp.bfloat16),
            jax.ShapeDtypeStruct((B, D), jnp.bfloat16),
            jax.ShapeDtypeStruct((B, D), jnp.bfloat16),
        ),
    )(gI, gT, W_i, W_t)

    grid = (B // BM,)
    blk = lambda i: (i, 0)
    out_shapes = (
        jax.ShapeDtypeStruct((B, B), jnp.float32),
        jax.ShapeDtypeStruct((B, D), jnp.float32),
        jax.ShapeDtypeStruct((B, D), jnp.float32),
    )
    S, gI2, gT2 = pl.pallas_call(
        _body,
        grid=grid,
        in_specs=[
            pl.BlockSpec((BM, D), blk),
            pl.BlockSpec((BM, D), blk),
            pl.BlockSpec((B, D), full),
            pl.BlockSpec((B, D), full),
            pl.BlockSpec((B, D), full),
            pl.BlockSpec((B, D), full),
            pl.BlockSpec((1, D), full),
            pl.BlockSpec((1, D), full),
            pl.BlockSpec((1, D), full),
            pl.BlockSpec((1, D), full),
        ],
        out_specs=(
            pl.BlockSpec((BM, B), blk),
            pl.BlockSpec((BM, D), blk),
            pl.BlockSpec((BM, D), blk),
        ),
        out_shape=out_shapes,
    )(gI, gT, gin, gtn, Pi, Pt,
      ln_i_g.reshape(1, D), ln_i_b.reshape(1, D),
      ln_t_g.reshape(1, D), ln_t_b.reshape(1, D))
    return (gI2, gT2, S)


# speedup vs baseline: 1.0330x; 1.0330x over previous
import functools

import jax
import jax.numpy as jnp
from jax.experimental import pallas as pl
from jax.experimental.pallas import tpu as pltpu

TAU = 0.2
TOPK = 8
ALPHA = 0.3
B = 4096
D = 128
BM = 512

_NEG = -3.0e38
CHUNK = 512


def _norm_rows(x):
    ss = jnp.sum(x * x, axis=1, keepdims=True)
    return x * jax.lax.rsqrt(jnp.maximum(ss, 1e-24))


def _layer_norm(y, gamma, beta):
    mu = jnp.mean(y, axis=1, keepdims=True)
    var = jnp.mean((y - mu) * (y - mu), axis=1, keepdims=True)
    return (y - mu) * jax.lax.rsqrt(var + 1e-5) * gamma + beta


_SORT8_STAGES = (
    ((0, 1), (2, 3), (4, 5), (6, 7)),
    ((0, 2), (1, 3), (4, 6), (5, 7)),
    ((1, 2), (5, 6)),
    ((0, 4), (1, 5), (2, 6), (3, 7)),
    ((2, 4), (3, 5)),
    ((1, 2), (3, 4), (5, 6)),
)
_BITONIC8_STAGES = (
    ((0, 4), (1, 5), (2, 6), (3, 7)),
    ((0, 2), (1, 3), (4, 6), (5, 7)),
    ((0, 1), (2, 3), (4, 5), (6, 7)),
)


def _apply_net(v, stages):
    v = list(v)
    for stage in stages:
        for i, j in stage:
            hi = jnp.maximum(v[i], v[j])
            lo = jnp.minimum(v[i], v[j])
            v[i], v[j] = hi, lo
    return v


def _merge_top8(a, b):
    d = [jnp.maximum(a[i], b[7 - i]) for i in range(8)]
    return _apply_net(d, _BITONIC8_STAGES)


def _topk_softmax_msg(s, P, g_raw, gamma, beta):
    ncol = s.shape[1] // 128
    thr_parts, z_parts = [], []
    for c in range(s.shape[0] // CHUNK):
        sc = s[c * CHUNK:(c + 1) * CHUNK, :]
        sl = [sc[:, k * 128:(k + 1) * 128] for k in range(ncol)]
        groups = [_apply_net(sl[8 * k:8 * k + 8], _SORT8_STAGES)
                  for k in range(ncol // 8)]
        while len(groups) > 1:
            groups = [_merge_top8(groups[2 * k], groups[2 * k + 1])
                      for k in range(len(groups) // 2)]
        d = groups[0]
        tops = []
        for _ in range(TOPK - 1):
            m = jnp.max(d[0], axis=1, keepdims=True)
            tops.append(m)
            eq = d[0] == m
            for i in range(TOPK - 1):
                d[i] = jnp.where(eq, d[i + 1], d[i])
            d[TOPK - 1] = jnp.where(eq, _NEG, d[TOPK - 1])
        thr_c = jnp.max(d[0], axis=1, keepdims=True)
        tops.append(thr_c)
        thr_parts.append(thr_c)
        z_parts.append(sum(
            jnp.exp(m.astype(jnp.bfloat16).astype(jnp.float32))
            for m in tops))
    thr = jnp.concatenate(thr_parts, axis=0)
    z = jnp.concatenate(z_parts, axis=0)
    e = jnp.where(s >= thr, jnp.exp(s.astype(jnp.bfloat16)),
                  jnp.bfloat16(0.0))
    msg = jax.lax.dot_general(
        e, P,
        (((1,), (0,)), ((), ())),
        preferred_element_type=jnp.float32,
    ) / z
    return _layer_norm(g_raw + ALPHA * msg, gamma, beta)


def _body(gI_blk, gT_blk, gI_full, gT_full, W_i, W_t,
          ln_i_g, ln_i_b, ln_t_g, ln_t_b,
          S_ref, gI2_ref, gT2_ref,
          gin, gtn, Pi, Pt):
    i = pl.program_id(0)

    @pl.when(i == 0)
    def _init():
        gI = gI_full[...]
        gT = gT_full[...]
        gin[...] = _norm_rows(gI).astype(jnp.bfloat16)
        gtn[...] = _norm_rows(gT).astype(jnp.bfloat16)
        Pi[...] = jax.lax.dot_general(
            gI.astype(jnp.bfloat16), W_i[...].astype(jnp.bfloat16),
            (((1,), (1,)), ((), ())),
            preferred_element_type=jnp.float32).astype(jnp.bfloat16)
        Pt[...] = jax.lax.dot_general(
            gT.astype(jnp.bfloat16), W_t[...].astype(jnp.bfloat16),
            (((1,), (1,)), ((), ())),
            preferred_element_type=jnp.float32).astype(jnp.bfloat16)

    inv_tau = jnp.float32(1.0 / TAU)

    gib = _norm_rows(gI_blk[...]).astype(jnp.bfloat16)
    s = jax.lax.dot_general(
        gib, gtn[...],
        (((1,), (1,)), ((), ())),
        preferred_element_type=jnp.float32) * inv_tau
    S_ref[...] = s
    gI2_ref[...] = _topk_softmax_msg(
        s, Pt[...], gI_blk[...], ln_i_g[...], ln_i_b[...])

    gtb = _norm_rows(gT_blk[...]).astype(jnp.bfloat16)
    st = jax.lax.dot_general(
        gtb, gin[...],
        (((1,), (1,)), ((), ())),
        preferred_element_type=jnp.float32) * inv_tau
    gT2_ref[...] = _topk_softmax_msg(
        st, Pi[...], gT_blk[...], ln_t_g[...], ln_t_b[...])


@jax.jit
def kernel(gI, gT, W_i, W_t, ln_i_g, ln_i_b, ln_t_g, ln_t_b):
    grid = (B // BM,)
    blk = lambda i: (i, 0)
    full = lambda i: (0, 0)
    out_shapes = (
        jax.ShapeDtypeStruct((B, B), jnp.float32),
        jax.ShapeDtypeStruct((B, D), jnp.float32),
        jax.ShapeDtypeStruct((B, D), jnp.float32),
    )
    S, gI2, gT2 = pl.pallas_call(
        _body,
        grid=grid,
        in_specs=[
            pl.BlockSpec((BM, D), blk),
            pl.BlockSpec((BM, D), blk),
            pl.BlockSpec((B, D), full),
            pl.BlockSpec((B, D), full),
            pl.BlockSpec((D, D), full),
            pl.BlockSpec((D, D), full),
            pl.BlockSpec((1, D), full),
            pl.BlockSpec((1, D), full),
            pl.BlockSpec((1, D), full),
            pl.BlockSpec((1, D), full),
        ],
        out_specs=(
            pl.BlockSpec((BM, B), blk),
            pl.BlockSpec((BM, D), blk),
            pl.BlockSpec((BM, D), blk),
        ),
        out_shape=out_shapes,
        scratch_shapes=[
            pltpu.VMEM((B, D), jnp.bfloat16),
            pltpu.VMEM((B, D), jnp.bfloat16),
            pltpu.VMEM((B, D), jnp.bfloat16),
            pltpu.VMEM((B, D), jnp.bfloat16),
        ],
    )(gI, gT, gI, gT, W_i, W_t,
      ln_i_g.reshape(1, D), ln_i_b.reshape(1, D),
      ln_t_g.reshape(1, D), ln_t_b.reshape(1, D))
    return (gI2, gT2, S)


# speedup vs baseline: 1.0358x; 1.0027x over previous
import jax
import jax.numpy as jnp
from jax.experimental import pallas as pl
from jax.experimental.pallas import tpu as pltpu

TAU = 0.2
TOPK = 8
ALPHA = 0.3
B = 4096
D = 128
BM = 512

_NEG = -3.0e38
CHUNK = 512


def _norm_rows(x):
    ss = jnp.sum(x * x, axis=1, keepdims=True)
    return x * jax.lax.rsqrt(jnp.maximum(ss, 1e-24))


def _layer_norm(y, gamma, beta):
    mu = jnp.mean(y, axis=1, keepdims=True)
    var = jnp.mean((y - mu) * (y - mu), axis=1, keepdims=True)
    return (y - mu) * jax.lax.rsqrt(var + 1e-5) * gamma + beta


_SORT8_STAGES = (
    ((0, 1), (2, 3), (4, 5), (6, 7)),
    ((0, 2), (1, 3), (4, 6), (5, 7)),
    ((1, 2), (5, 6)),
    ((0, 4), (1, 5), (2, 6), (3, 7)),
    ((2, 4), (3, 5)),
    ((1, 2), (3, 4), (5, 6)),
)
_BITONIC8_STAGES = (
    ((0, 4), (1, 5), (2, 6), (3, 7)),
    ((0, 2), (1, 3), (4, 6), (5, 7)),
    ((0, 1), (2, 3), (4, 5), (6, 7)),
)


def _apply_net(v, stages):
    v = list(v)
    for stage in stages:
        for i, j in stage:
            hi = jnp.maximum(v[i], v[j])
            lo = jnp.minimum(v[i], v[j])
            v[i], v[j] = hi, lo
    return v


def _merge_top8(a, b):
    d = [jnp.maximum(a[i], b[7 - i]) for i in range(8)]
    return _apply_net(d, _BITONIC8_STAGES)


def _topk_softmax_msg(s, P, g_raw, gamma, beta):
    ncol = s.shape[1] // 128
    thr_parts, z_parts = [], []
    for c in range(s.shape[0] // CHUNK):
        sc = s[c * CHUNK:(c + 1) * CHUNK, :]
        sl = [sc[:, k * 128:(k + 1) * 128] for k in range(ncol)]
        groups = [_apply_net(sl[8 * k:8 * k + 8], _SORT8_STAGES)
                  for k in range(ncol // 8)]
        while len(groups) > 1:
            groups = [_merge_top8(groups[2 * k], groups[2 * k + 1])
                      for k in range(len(groups) // 2)]
        d = groups[0]
        tops = []
        for _ in range(TOPK - 1):
            m = jnp.max(d[0], axis=1, keepdims=True)
            tops.append(m)
            eq = d[0] == m
            for i in range(TOPK - 1):
                d[i] = jnp.where(eq, d[i + 1], d[i])
            d[TOPK - 1] = jnp.where(eq, _NEG, d[TOPK - 1])
        thr_c = jnp.max(d[0], axis=1, keepdims=True)
        tops.append(thr_c)
        thr_parts.append(thr_c)
        z_parts.append(sum(
            jnp.exp(m.astype(jnp.bfloat16).astype(jnp.float32))
            for m in tops))
    thr = jnp.concatenate(thr_parts, axis=0)
    z = jnp.concatenate(z_parts, axis=0)
    e = jnp.where(s >= thr, jnp.exp(s.astype(jnp.bfloat16)),
                  jnp.bfloat16(0.0))
    msg = jax.lax.dot_general(
        e, P,
        (((1,), (0,)), ((), ())),
        preferred_element_type=jnp.float32,
    ) / z
    return _layer_norm(g_raw + ALPHA * msg, gamma, beta)


def _body(gI_blk, gT_blk, gI_full, gT_full, W_i, W_t,
          ln_i_g, ln_i_b, ln_t_g, ln_t_b,
          S_ref, gI2_ref, gT2_ref,
          gin, gtn, Pi, Pt):
    i = pl.program_id(0)

    @pl.when(i == 0)
    def _init():
        gI = gI_full[...]
        gT = gT_full[...]
        gin[...] = _norm_rows(gI).astype(jnp.bfloat16)
        gtn[...] = _norm_rows(gT).astype(jnp.bfloat16)
        Pi[...] = jax.lax.dot_general(
            gI.astype(jnp.bfloat16), W_i[...].astype(jnp.bfloat16),
            (((1,), (1,)), ((), ())),
            preferred_element_type=jnp.float32).astype(jnp.bfloat16)
        Pt[...] = jax.lax.dot_general(
            gT.astype(jnp.bfloat16), W_t[...].astype(jnp.bfloat16),
            (((1,), (1,)), ((), ())),
            preferred_element_type=jnp.float32).astype(jnp.bfloat16)

    inv_tau = jnp.float32(1.0 / TAU)

    gib = _norm_rows(gI_blk[...]).astype(jnp.bfloat16)
    s = jax.lax.dot_general(
        gib, gtn[...],
        (((1,), (1,)), ((), ())),
        preferred_element_type=jnp.float32) * inv_tau
    S_ref[...] = s
    gI2_ref[...] = _topk_softmax_msg(
        s, Pt[...], gI_blk[...], ln_i_g[...], ln_i_b[...])

    gtb = _norm_rows(gT_blk[...]).astype(jnp.bfloat16)
    st = jax.lax.dot_general(
        gtb, gin[...],
        (((1,), (1,)), ((), ())),
        preferred_element_type=jnp.float32) * inv_tau
    gT2_ref[...] = _topk_softmax_msg(
        st, Pi[...], gT_blk[...], ln_t_g[...], ln_t_b[...])


@jax.jit
def kernel(gI, gT, W_i, W_t, ln_i_g, ln_i_b, ln_t_g, ln_t_b):
    grid = (B // BM,)
    blk = lambda i: (i, 0)
    full = lambda i: (0, 0)
    out_shapes = (
        jax.ShapeDtypeStruct((B, B), jnp.float32),
        jax.ShapeDtypeStruct((B, D), jnp.float32),
        jax.ShapeDtypeStruct((B, D), jnp.float32),
    )
    S, gI2, gT2 = pl.pallas_call(
        _body,
        grid=grid,
        in_specs=[
            pl.BlockSpec((BM, D), blk),
            pl.BlockSpec((BM, D), blk),
            pl.BlockSpec((B, D), full),
            pl.BlockSpec((B, D), full),
            pl.BlockSpec((D, D), full),
            pl.BlockSpec((D, D), full),
            pl.BlockSpec((1, D), full),
            pl.BlockSpec((1, D), full),
            pl.BlockSpec((1, D), full),
            pl.BlockSpec((1, D), full),
        ],
        out_specs=(
            pl.BlockSpec((BM, B), blk),
            pl.BlockSpec((BM, D), blk),
            pl.BlockSpec((BM, D), blk),
        ),
        out_shape=out_shapes,
        scratch_shapes=[
            pltpu.VMEM((B, D), jnp.bfloat16),
            pltpu.VMEM((B, D), jnp.bfloat16),
            pltpu.VMEM((B, D), jnp.bfloat16),
            pltpu.VMEM((B, D), jnp.bfloat16),
        ],
    )(gI, gT, gI, gT, W_i, W_t,
      ln_i_g.reshape(1, D), ln_i_b.reshape(1, D),
      ln_t_g.reshape(1, D), ln_t_b.reshape(1, D))
    return (gI2, gT2, S)


# speedup vs baseline: 1.0665x; 1.0296x over previous
import jax
import jax.numpy as jnp
from jax.experimental import pallas as pl
from jax.experimental.pallas import tpu as pltpu

TAU = 0.2
TOPK = 8
ALPHA = 0.3
B = 4096
D = 128
BM = 512

_NEG = -3.0e38
CHUNK = 512


def _norm_rows(x):
    ss = jnp.sum(x * x, axis=1, keepdims=True)
    return x * jax.lax.rsqrt(jnp.maximum(ss, 1e-24))


def _layer_norm(y, gamma, beta):
    mu = jnp.mean(y, axis=1, keepdims=True)
    var = jnp.mean((y - mu) * (y - mu), axis=1, keepdims=True)
    return (y - mu) * jax.lax.rsqrt(var + 1e-5) * gamma + beta


_SORT8_STAGES = (
    ((0, 1), (2, 3), (4, 5), (6, 7)),
    ((0, 2), (1, 3), (4, 6), (5, 7)),
    ((1, 2), (5, 6)),
    ((0, 4), (1, 5), (2, 6), (3, 7)),
    ((2, 4), (3, 5)),
    ((1, 2), (3, 4), (5, 6)),
)
_BITONIC8_STAGES = (
    ((0, 4), (1, 5), (2, 6), (3, 7)),
    ((0, 2), (1, 3), (4, 6), (5, 7)),
    ((0, 1), (2, 3), (4, 5), (6, 7)),
)


def _apply_net(v, stages):
    v = list(v)
    for stage in stages:
        for i, j in stage:
            hi = jnp.maximum(v[i], v[j])
            lo = jnp.minimum(v[i], v[j])
            v[i], v[j] = hi, lo
    return v


def _merge_top8(a, b):
    d = [jnp.maximum(a[i], b[7 - i]) for i in range(8)]
    return _apply_net(d, _BITONIC8_STAGES)


def _topk_softmax_msg(s, P, g_raw, gamma, beta):
    ncol = s.shape[1] // 128
    thr_parts, z_parts = [], []
    for c in range(s.shape[0] // CHUNK):
        sc = s[c * CHUNK:(c + 1) * CHUNK, :]
        sl = [sc[:, k * 128:(k + 1) * 128] for k in range(ncol)]
        groups = [_apply_net(sl[8 * k:8 * k + 8], _SORT8_STAGES)
                  for k in range(ncol // 8)]
        while len(groups) > 1:
            groups = [_merge_top8(groups[2 * k], groups[2 * k + 1])
                      for k in range(len(groups) // 2)]
        d = groups[0]
        tops = []
        for _ in range(TOPK - 1):
            m = jnp.max(d[0], axis=1, keepdims=True)
            tops.append(m)
            eq = d[0] == m
            for i in range(TOPK - 1):
                d[i] = jnp.where(eq, d[i + 1], d[i])
            d[TOPK - 1] = jnp.where(eq, _NEG, d[TOPK - 1])
        thr_c = jnp.max(d[0], axis=1, keepdims=True)
        tops.append(thr_c)
        thr_parts.append(thr_c)
        z_parts.append(sum(
            jnp.exp(m.astype(jnp.bfloat16).astype(jnp.float32))
            for m in tops))
    thr = jnp.concatenate(thr_parts, axis=0)
    z = jnp.concatenate(z_parts, axis=0)
    e = jnp.where(s >= thr, jnp.exp(s.astype(jnp.bfloat16)),
                  jnp.bfloat16(0.0))
    msg = jax.lax.dot_general(
        e, P,
        (((1,), (0,)), ((), ())),
        preferred_element_type=jnp.float32,
    ) / z
    return _layer_norm(g_raw + ALPHA * msg, gamma, beta)


def _body(gI_blk, gT_blk, gI_full, gT_full, W_i, W_t,
          ln_i_g, ln_i_b, ln_t_g, ln_t_b,
          S_ref, gI2_ref, gT2_ref,
          gin, gtn, Pi, Pt):
    i = pl.program_id(0)

    @pl.when(i == 0)
    def _init():
        gI = gI_full[...]
        gT = gT_full[...]
        gin[...] = _norm_rows(gI).astype(jnp.bfloat16)
        gtn[...] = _norm_rows(gT).astype(jnp.bfloat16)
        Pi[...] = jax.lax.dot_general(
            gI.astype(jnp.bfloat16), W_i[...].astype(jnp.bfloat16),
            (((1,), (1,)), ((), ())),
            preferred_element_type=jnp.float32).astype(jnp.bfloat16)
        Pt[...] = jax.lax.dot_general(
            gT.astype(jnp.bfloat16), W_t[...].astype(jnp.bfloat16),
            (((1,), (1,)), ((), ())),
            preferred_element_type=jnp.float32).astype(jnp.bfloat16)

    inv_tau = jnp.float32(1.0 / TAU)

    gib = _norm_rows(gI_blk[...]).astype(jnp.bfloat16)
    s = jax.lax.dot_general(
        gib, gtn[...],
        (((1,), (1,)), ((), ())),
        preferred_element_type=jnp.float32) * inv_tau
    S_ref[...] = s
    gtb = _norm_rows(gT_blk[...]).astype(jnp.bfloat16)
    st = jax.lax.dot_general(
        gtb, gin[...],
        (((1,), (1,)), ((), ())),
        preferred_element_type=jnp.float32) * inv_tau
    gI2_ref[...] = _topk_softmax_msg(
        s, Pt[...], gI_blk[...], ln_i_g[...], ln_i_b[...])
    gT2_ref[...] = _topk_softmax_msg(
        st, Pi[...], gT_blk[...], ln_t_g[...], ln_t_b[...])


@jax.jit
def kernel(gI, gT, W_i, W_t, ln_i_g, ln_i_b, ln_t_g, ln_t_b):
    grid = (B // BM,)
    blk = lambda i: (i, 0)
    full = lambda i: (0, 0)
    out_shapes = (
        jax.ShapeDtypeStruct((B, B), jnp.float32),
        jax.ShapeDtypeStruct((B, D), jnp.float32),
        jax.ShapeDtypeStruct((B, D), jnp.float32),
    )
    S, gI2, gT2 = pl.pallas_call(
        _body,
        grid=grid,
        in_specs=[
            pl.BlockSpec((BM, D), blk),
            pl.BlockSpec((BM, D), blk),
            pl.BlockSpec((B, D), full),
            pl.BlockSpec((B, D), full),
            pl.BlockSpec((D, D), full),
            pl.BlockSpec((D, D), full),
            pl.BlockSpec((1, D), full),
            pl.BlockSpec((1, D), full),
            pl.BlockSpec((1, D), full),
            pl.BlockSpec((1, D), full),
        ],
        out_specs=(
            pl.BlockSpec((BM, B), blk),
            pl.BlockSpec((BM, D), blk),
            pl.BlockSpec((BM, D), blk),
        ),
        out_shape=out_shapes,
        scratch_shapes=[
            pltpu.VMEM((B, D), jnp.bfloat16),
            pltpu.VMEM((B, D), jnp.bfloat16),
            pltpu.VMEM((B, D), jnp.bfloat16),
            pltpu.VMEM((B, D), jnp.bfloat16),
        ],
    )(gI, gT, gI, gT, W_i, W_t,
      ln_i_g.reshape(1, D), ln_i_b.reshape(1, D),
      ln_t_g.reshape(1, D), ln_t_b.reshape(1, D))
    return (gI2, gT2, S)
